# Initial kernel scaffold; baseline (speedup 1.0000x reference)
import jax
import jax.numpy as jnp
from jax.experimental import pallas as pl


def kernel(z_e, codebook):
    raise NotImplementedError("write your pallas kernel here")


# speedup vs baseline: 2.2252x; 2.2252x over previous
import jax
import jax.numpy as jnp
from jax.experimental import pallas as pl


def _vq_loss_kernel(z_ref, cb_ref, loss_ref):
    b = pl.program_id(0)
    z = z_ref[0]
    cb = cb_ref[...]

    dots = jax.lax.dot_general(
        z, cb, (((0,), (1,)), ((), ())), preferred_element_type=jnp.float32
    )
    c_norms = jnp.sum(cb * cb, axis=1)
    z_norms = jnp.sum(z * z, axis=0)
    dists = z_norms[:, None] - 2.0 * dots + c_norms[None, :]
    partial = jnp.sum(jnp.min(dists, axis=1)).reshape(1, 1)

    @pl.when(b == 0)
    def _init():
        loss_ref[...] = jnp.zeros_like(loss_ref)

    loss_ref[...] += partial


def kernel(z_e, codebook):
    B, D, H, W = z_e.shape
    HW = H * W
    K = codebook.shape[0]
    z2 = z_e.reshape(B, D, HW)

    loss_sum = pl.pallas_call(
        _vq_loss_kernel,
        grid=(B,),
        in_specs=[
            pl.BlockSpec((1, D, HW), lambda b: (b, 0, 0)),
            pl.BlockSpec((K, D), lambda b: (0, 0)),
        ],
        out_specs=pl.BlockSpec((1, 1), lambda b: (0, 0)),
        out_shape=jax.ShapeDtypeStruct((1, 1), jnp.float32),
    )(z2, codebook)

    vq_loss = (2.0 / z_e.size) * loss_sum[0, 0]
    return z_e, vq_loss


# speedup vs baseline: 2.3837x; 1.0712x over previous
import jax
import jax.numpy as jnp
from jax.experimental import pallas as pl


def _vq_loss_kernel(z_ref, cb_ref, loss_ref):
    b = pl.program_id(0)
    z = z_ref[0]
    cb = cb_ref[...]

    cbs = (-2.0 * cb).astype(jnp.bfloat16)
    dots = jax.lax.dot_general(
        cbs, z.astype(jnp.bfloat16),
        (((1,), (0,)), ((), ())), preferred_element_type=jnp.float32,
    )
    c_norms = jnp.sum(cb * cb, axis=1)
    z_norms = jnp.sum(z * z, axis=0)
    min_d = jnp.min(dots + c_norms[:, None], axis=0)
    partial = jnp.sum(min_d + z_norms).reshape(1, 1)

    @pl.when(b == 0)
    def _init():
        loss_ref[...] = jnp.zeros_like(loss_ref)

    loss_ref[...] += partial


def kernel(z_e, codebook):
    B, D, H, W = z_e.shape
    HW = H * W
    K = codebook.shape[0]
    z2 = z_e.reshape(B, D, HW)

    loss_sum = pl.pallas_call(
        _vq_loss_kernel,
        grid=(B,),
        in_specs=[
            pl.BlockSpec((1, D, HW), lambda b: (b, 0, 0)),
            pl.BlockSpec((K, D), lambda b: (0, 0)),
        ],
        out_specs=pl.BlockSpec((1, 1), lambda b: (0, 0)),
        out_shape=jax.ShapeDtypeStruct((1, 1), jnp.float32),
    )(z2, codebook)

    vq_loss = (2.0 / z_e.size) * loss_sum[0, 0]
    return z_e, vq_loss
